# Initial kernel scaffold; baseline (speedup 1.0000x reference)
import jax
import jax.numpy as jnp
from jax.experimental import pallas as pl


def kernel(x, edge_index, edge_attr, W1, a_src1, a_dst1, We1, ae1, b1, W2, a_src2, a_dst2, We2, ae2, b2):
    raise NotImplementedError("write your pallas kernel here")


# speedup vs baseline: 10.5947x; 10.5947x over previous
import functools

import jax
import jax.numpy as jnp
from jax import lax
from jax.experimental import pallas as pl
from jax.experimental.pallas import tpu as pltpu
from jax.experimental.pallas import tpu_sc as plsc

_N = 10000
_NP = 10240
_E = 320000
_D = 128
_DE = 16
_NC = 2
_NS = 16
_NT = _NC * _NS
_CH = 128
_NCHUNK = 80
_EP = _NT * _NCHUNK * _CH
_RPT = _NP // _NS


def _tc_edge_alpha_body(eap_ref, wbig_ref, eal_ref, csum_ref):
  i = pl.program_id(0)
  xb = eap_ref[...]
  eal_ref[...] = jnp.dot(xb, wbig_ref[...],
                         preferred_element_type=jnp.float32)
  part = jnp.broadcast_to(jnp.sum(xb, axis=0, keepdims=True), (8, _D))

  @pl.when(i == 0)
  def _():
    csum_ref[...] = part

  @pl.when(i > 0)
  def _():
    csum_ref[...] = csum_ref[...] + part


def _edge_alpha(eap, wbig):
  g = 10
  rows = eap.shape[0] // g
  return pl.pallas_call(
      _tc_edge_alpha_body,
      grid=(g,),
      in_specs=[
          pl.BlockSpec((rows, _D), lambda i: (i, 0)),
          pl.BlockSpec((_D, _D), lambda i: (0, 0)),
      ],
      out_specs=[
          pl.BlockSpec((rows, _D), lambda i: (i, 0)),
          pl.BlockSpec((8, _D), lambda i: (0, 0)),
      ],
      out_shape=[
          jax.ShapeDtypeStruct((eap.shape[0], _D), jnp.float32),
          jax.ShapeDtypeStruct((8, _D), jnp.float32),
      ],
  )(eap, wbig)


def _proj_tail(h, aa_ref, elo_ref, hs_ref, hd_ref, exs_ref):
  hs = jnp.sum(h * aa_ref[0:1, :], axis=1, keepdims=True)
  hd = jnp.sum(h * aa_ref[1:2, :], axis=1, keepdims=True)
  hs_ref[...] = hs
  hd_ref[...] = hd
  al = hs + hd + elo_ref[0]
  al = jnp.where(al >= 0.0, al, 0.2 * al)
  exs_ref[...] = jnp.exp(al)


def _tc_dense1_body(x_ref, w_ref, aa_ref, elo_ref, h_ref, hs_ref, hd_ref,
                    exs_ref):
  h = jnp.dot(x_ref[...], w_ref[...], preferred_element_type=jnp.float32)
  h_ref[...] = h
  _proj_tail(h, aa_ref, elo_ref, hs_ref, hd_ref, exs_ref)


def _dense1(x, w, aa, elo):
  g = 10
  b = _NP // g
  return pl.pallas_call(
      _tc_dense1_body,
      grid=(g,),
      in_specs=[
          pl.BlockSpec((b, _D), lambda i: (i, 0)),
          pl.BlockSpec((_D, _D), lambda i: (0, 0)),
          pl.BlockSpec((8, _D), lambda i: (0, 0)),
          pl.BlockSpec(memory_space=pltpu.SMEM),
      ],
      out_specs=[
          pl.BlockSpec((b, _D), lambda i: (i, 0)),
          pl.BlockSpec((b, 1), lambda i: (i, 0)),
          pl.BlockSpec((b, 1), lambda i: (i, 0)),
          pl.BlockSpec((b, 1), lambda i: (i, 0)),
      ],
      out_shape=[
          jax.ShapeDtypeStruct((_NP, _D), jnp.float32),
          jax.ShapeDtypeStruct((_NP, 1), jnp.float32),
          jax.ShapeDtypeStruct((_NP, 1), jnp.float32),
          jax.ShapeDtypeStruct((_NP, 1), jnp.float32),
      ],
  )(x, w, aa, elo)


def _combine_num_den(acc_ref, den_ref, exs_ref, h_ref, b_ref):
  exs = exs_ref[...]
  num = acc_ref[0] + acc_ref[1] + exs * h_ref[...]
  den = jnp.sum(den_ref[...], axis=0) + exs
  return num / jnp.maximum(den, 1e-16) + b_ref[0:1, :]


def _tc_mid_body(acc_ref, den_ref, exs_ref, h_ref, b_ref, w_ref, aa_ref,
                 elo_ref, h2_ref, hs_ref, hd_ref, exs_ref_o):
  x2 = jnp.maximum(_combine_num_den(acc_ref, den_ref, exs_ref, h_ref, b_ref),
                   0.0)
  h2 = jnp.dot(x2, w_ref[...], preferred_element_type=jnp.float32)
  h2_ref[...] = h2
  _proj_tail(h2, aa_ref, elo_ref, hs_ref, hd_ref, exs_ref_o)


def _mid(acc, den, exs, h, bias, w, aa, elo):
  g = 10
  b = _NP // g
  return pl.pallas_call(
      _tc_mid_body,
      grid=(g,),
      in_specs=[
          pl.BlockSpec((2, b, _D), lambda i: (0, i, 0)),
          pl.BlockSpec((_NT, b, 1), lambda i: (0, i, 0)),
          pl.BlockSpec((b, 1), lambda i: (i, 0)),
          pl.BlockSpec((b, _D), lambda i: (i, 0)),
          pl.BlockSpec((8, _D), lambda i: (0, 0)),
          pl.BlockSpec((_D, _D), lambda i: (0, 0)),
          pl.BlockSpec((8, _D), lambda i: (0, 0)),
          pl.BlockSpec(memory_space=pltpu.SMEM),
      ],
      out_specs=[
          pl.BlockSpec((b, _D), lambda i: (i, 0)),
          pl.BlockSpec((b, 1), lambda i: (i, 0)),
          pl.BlockSpec((b, 1), lambda i: (i, 0)),
          pl.BlockSpec((b, 1), lambda i: (i, 0)),
      ],
      out_shape=[
          jax.ShapeDtypeStruct((_NP, _D), jnp.float32),
          jax.ShapeDtypeStruct((_NP, 1), jnp.float32),
          jax.ShapeDtypeStruct((_NP, 1), jnp.float32),
          jax.ShapeDtypeStruct((_NP, 1), jnp.float32),
      ],
  )(acc, den, exs, h, bias, w, aa, elo)


def _tc_final_body(acc_ref, den_ref, exs_ref, h_ref, b_ref, out_ref):
  out_ref[...] = _combine_num_den(acc_ref, den_ref, exs_ref, h_ref, b_ref)


def _final(acc, den, exs, h, bias):
  g = 10
  b = _NP // g
  return pl.pallas_call(
      _tc_final_body,
      grid=(g,),
      in_specs=[
          pl.BlockSpec((2, b, _D), lambda i: (0, i, 0)),
          pl.BlockSpec((_NT, b, 1), lambda i: (0, i, 0)),
          pl.BlockSpec((b, 1), lambda i: (i, 0)),
          pl.BlockSpec((b, _D), lambda i: (i, 0)),
          pl.BlockSpec((8, _D), lambda i: (0, 0)),
      ],
      out_specs=pl.BlockSpec((b, _D), lambda i: (i, 0)),
      out_shape=jax.ShapeDtypeStruct((_NP, _D), jnp.float32),
  )(acc, den, exs, h, bias)


def _sc_edge_body(h_hbm, hs_hbm, hd_hbm, src_hbm, dst_hbm, eal_hbm,
                  acc_hbm, den_hbm,
                  hs_v, hd_v, den_v, src_v, dst_v, eal_v, ex_v, rows_v,
                  acc_s, sem):
  cid = lax.axis_index("c")
  sid = lax.axis_index("s")
  tid = cid * _NS + sid

  pltpu.sync_copy(hs_hbm, hs_v)
  pltpu.sync_copy(hd_hbm, hd_v)

  zero16 = jnp.zeros((16,), jnp.float32)

  def _zero_den(i, c):
    den_v[pl.ds(i * 16, 16)] = zero16
    return c
  lax.fori_loop(0, _NP // 16, _zero_den, 0)

  def _zero_rows(i, c):
    for q in range(_D // 16):
      rows_v[0, i, pl.ds(q * 16, 16)] = zero16
    return c
  lax.fori_loop(0, _CH, _zero_rows, 0)
  for k in range(_RPT // _CH):
    pltpu.sync_copy(rows_v.at[0],
                    acc_s.at[pl.ds(sid * _RPT + k * _CH, _CH)])
  plsc.subcore_barrier()

  def _chunk(j, c):
    b = 0
    pltpu.sync_copy(src_hbm.at[tid, j], src_v.at[b])
    pltpu.sync_copy(dst_hbm.at[tid, j], dst_v.at[b])
    pltpu.sync_copy(eal_hbm.at[tid, j], eal_v.at[b])
    gat = pltpu.async_copy(h_hbm.at[src_v.at[b]], rows_v.at[b], sem)
    for q in range(_CH // 16):
      s16 = src_v[b, pl.ds(q * 16, 16)]
      d16 = dst_v[b, pl.ds(q * 16, 16)]
      al = (plsc.load_gather(hs_v, [s16]) + plsc.load_gather(hd_v, [d16])
            + eal_v[b, pl.ds(q * 16, 16)])
      al = jnp.where(al >= 0.0, al, 0.2 * al)
      e16 = jnp.exp(al)
      ex_v[b, pl.ds(q * 16, 16)] = e16
      plsc.addupdate_scatter(den_v, [d16], e16)
    gat.wait()

    def _scale(g, c2):
      e16 = ex_v[b, pl.ds(g * 16, 16)]
      for l in range(16):
        i = g * 16 + l
        e = e16[l]
        for q in range(_D // 16):
          rows_v[b, i, pl.ds(q * 16, 16)] = rows_v[b, i, pl.ds(q * 16, 16)] * e
      return c2
    lax.fori_loop(0, _CH // 16, _scale, 0)
    pltpu.sync_copy(rows_v.at[b], acc_s.at[dst_v.at[b]], add=True)
    return c

  lax.fori_loop(0, _NCHUNK, _chunk, 0)

  pltpu.sync_copy(den_v, den_hbm.at[tid])
  plsc.subcore_barrier()
  for k in range(_RPT // _CH):
    r0 = sid * _RPT + k * _CH
    pltpu.sync_copy(acc_s.at[pl.ds(r0, _CH)], acc_hbm.at[cid, pl.ds(r0, _CH)])


_sc_edges = functools.partial(
    pl.kernel,
    out_type=[
        jax.ShapeDtypeStruct((_NC, _NP, _D), jnp.float32),
        jax.ShapeDtypeStruct((_NT, _NP), jnp.float32),
    ],
    mesh=plsc.VectorSubcoreMesh(core_axis_name="c", subcore_axis_name="s"),
    compiler_params=pltpu.CompilerParams(needs_layout_passes=False),
    scratch_types=[
        pltpu.VMEM((_NP,), jnp.float32),
        pltpu.VMEM((_NP,), jnp.float32),
        pltpu.VMEM((_NP,), jnp.float32),
        pltpu.VMEM((1, _CH), jnp.int32),
        pltpu.VMEM((1, _CH), jnp.int32),
        pltpu.VMEM((1, _CH), jnp.float32),
        pltpu.VMEM((1, _CH), jnp.float32),
        pltpu.VMEM((1, _CH, _D), jnp.float32),
        pltpu.VMEM_SHARED((_NP, _D), jnp.float32),
        pltpu.SemaphoreType.DMA,
    ],
)(_sc_edge_body)


@jax.jit
def kernel(x, edge_index, edge_attr, W1, a_src1, a_dst1, We1, ae1, b1,
           W2, a_src2, a_dst2, We2, ae2, b2):
  f32 = jnp.float32

  xp = jnp.pad(x, ((0, _NP - _N), (0, 0)))
  src = jnp.pad(edge_index[0], (0, _EP - _E)).reshape(_NT, _NCHUNK, _CH)
  dst = jnp.pad(edge_index[1], (0, _EP - _E)).reshape(_NT, _NCHUNK, _CH)
  eap = edge_attr.reshape(_E // 8, _D)

  w12 = jnp.stack([We1 @ ae1, We2 @ ae2], axis=1)
  wbig = jnp.pad(jnp.kron(jnp.eye(8, dtype=f32), w12), ((0, 0), (0, _D - 16)))

  ealp, csum = _edge_alpha(eap, wbig)
  eal2 = ealp[:, :16].reshape(_E, 2)
  ea_mean = csum[0].reshape(8, _DE).sum(axis=0) / float(_E)
  elo = ea_mean @ w12

  def pack_eal(l):
    v = jnp.pad(eal2[:, l], (0, _EP - _E), constant_values=-1e30)
    return v.reshape(_NT, _NCHUNK, _CH)

  aa1 = jnp.pad(jnp.stack([a_src1, a_dst1]), ((0, 6), (0, 0)))
  aa2 = jnp.pad(jnp.stack([a_src2, a_dst2]), ((0, 6), (0, 0)))
  b1p = jnp.pad(b1.reshape(1, _D), ((0, 7), (0, 0)))
  b2p = jnp.pad(b2.reshape(1, _D), ((0, 7), (0, 0)))

  h1, hs1, hd1, exs1 = _dense1(xp, W1, aa1, elo[0:1])
  acc1, den1 = _sc_edges(h1, hs1.reshape(_NP), hd1.reshape(_NP),
                         src, dst, pack_eal(0))
  den1r = den1.reshape(_NT, _NP, 1)

  h2, hs2, hd2, exs2 = _mid(acc1, den1r, exs1, h1, b1p, W2, aa2, elo[1:2])
  acc2, den2 = _sc_edges(h2, hs2.reshape(_NP), hd2.reshape(_NP),
                         src, dst, pack_eal(1))
  den2r = den2.reshape(_NT, _NP, 1)

  out = _final(acc2, den2r, exs2, h2, b2p)
  return out[:_N]


# speedup vs baseline: 12.6592x; 1.1949x over previous
import functools

import jax
import jax.numpy as jnp
from jax import lax
from jax.experimental import pallas as pl
from jax.experimental.pallas import tpu as pltpu
from jax.experimental.pallas import tpu_sc as plsc

_N = 10000
_NP = 10240
_E = 320000
_D = 128
_DE = 16
_NC = 2
_NS = 16
_NT = _NC * _NS
_CH = 64
_NCHUNK = 160
_EP = _NT * _NCHUNK * _CH
_RPT = _NP // _NS


def _tc_edge_alpha_body(eap_ref, wbig_ref, eal_ref, csum_ref):
  i = pl.program_id(0)
  xb = eap_ref[...]
  eal_ref[...] = jnp.dot(xb, wbig_ref[...],
                         preferred_element_type=jnp.float32)
  part = jnp.broadcast_to(jnp.sum(xb, axis=0, keepdims=True), (8, _D))

  @pl.when(i == 0)
  def _():
    csum_ref[...] = part

  @pl.when(i > 0)
  def _():
    csum_ref[...] = csum_ref[...] + part


def _edge_alpha(eap, wbig):
  g = 10
  rows = eap.shape[0] // g
  return pl.pallas_call(
      _tc_edge_alpha_body,
      grid=(g,),
      in_specs=[
          pl.BlockSpec((rows, _D), lambda i: (i, 0)),
          pl.BlockSpec((_D, _D), lambda i: (0, 0)),
      ],
      out_specs=[
          pl.BlockSpec((rows, _D), lambda i: (i, 0)),
          pl.BlockSpec((8, _D), lambda i: (0, 0)),
      ],
      out_shape=[
          jax.ShapeDtypeStruct((eap.shape[0], _D), jnp.float32),
          jax.ShapeDtypeStruct((8, _D), jnp.float32),
      ],
  )(eap, wbig)


def _proj_tail(h, aa_ref, elo_ref, hs_ref, hd_ref, exs_ref):
  hs = jnp.sum(h * aa_ref[0:1, :], axis=1, keepdims=True)
  hd = jnp.sum(h * aa_ref[1:2, :], axis=1, keepdims=True)
  hs_ref[...] = hs
  hd_ref[...] = hd
  al = hs + hd + elo_ref[0]
  al = jnp.where(al >= 0.0, al, 0.2 * al)
  exs_ref[...] = jnp.exp(al)


def _tc_dense1_body(x_ref, w_ref, aa_ref, elo_ref, h_ref, hs_ref, hd_ref,
                    exs_ref):
  h = jnp.dot(x_ref[...], w_ref[...], preferred_element_type=jnp.float32)
  h_ref[...] = h
  _proj_tail(h, aa_ref, elo_ref, hs_ref, hd_ref, exs_ref)


def _dense1(x, w, aa, elo):
  g = 10
  b = _NP // g
  return pl.pallas_call(
      _tc_dense1_body,
      grid=(g,),
      in_specs=[
          pl.BlockSpec((b, _D), lambda i: (i, 0)),
          pl.BlockSpec((_D, _D), lambda i: (0, 0)),
          pl.BlockSpec((8, _D), lambda i: (0, 0)),
          pl.BlockSpec(memory_space=pltpu.SMEM),
      ],
      out_specs=[
          pl.BlockSpec((b, _D), lambda i: (i, 0)),
          pl.BlockSpec((b, 1), lambda i: (i, 0)),
          pl.BlockSpec((b, 1), lambda i: (i, 0)),
          pl.BlockSpec((b, 1), lambda i: (i, 0)),
      ],
      out_shape=[
          jax.ShapeDtypeStruct((_NP, _D), jnp.float32),
          jax.ShapeDtypeStruct((_NP, 1), jnp.float32),
          jax.ShapeDtypeStruct((_NP, 1), jnp.float32),
          jax.ShapeDtypeStruct((_NP, 1), jnp.float32),
      ],
  )(x, w, aa, elo)


def _combine_num_den(acc_ref, den_ref, exs_ref, h_ref, b_ref):
  exs = exs_ref[...]
  num = acc_ref[0] + acc_ref[1] + exs * h_ref[...]
  den = jnp.sum(den_ref[...], axis=0) + exs
  return num / jnp.maximum(den, 1e-16) + b_ref[0:1, :]


def _tc_mid_body(acc_ref, den_ref, exs_ref, h_ref, b_ref, w_ref, aa_ref,
                 elo_ref, h2_ref, hs_ref, hd_ref, exs_ref_o):
  x2 = jnp.maximum(_combine_num_den(acc_ref, den_ref, exs_ref, h_ref, b_ref),
                   0.0)
  h2 = jnp.dot(x2, w_ref[...], preferred_element_type=jnp.float32)
  h2_ref[...] = h2
  _proj_tail(h2, aa_ref, elo_ref, hs_ref, hd_ref, exs_ref_o)


def _mid(acc, den, exs, h, bias, w, aa, elo):
  g = 10
  b = _NP // g
  return pl.pallas_call(
      _tc_mid_body,
      grid=(g,),
      in_specs=[
          pl.BlockSpec((2, b, _D), lambda i: (0, i, 0)),
          pl.BlockSpec((_NT, b, 1), lambda i: (0, i, 0)),
          pl.BlockSpec((b, 1), lambda i: (i, 0)),
          pl.BlockSpec((b, _D), lambda i: (i, 0)),
          pl.BlockSpec((8, _D), lambda i: (0, 0)),
          pl.BlockSpec((_D, _D), lambda i: (0, 0)),
          pl.BlockSpec((8, _D), lambda i: (0, 0)),
          pl.BlockSpec(memory_space=pltpu.SMEM),
      ],
      out_specs=[
          pl.BlockSpec((b, _D), lambda i: (i, 0)),
          pl.BlockSpec((b, 1), lambda i: (i, 0)),
          pl.BlockSpec((b, 1), lambda i: (i, 0)),
          pl.BlockSpec((b, 1), lambda i: (i, 0)),
      ],
      out_shape=[
          jax.ShapeDtypeStruct((_NP, _D), jnp.float32),
          jax.ShapeDtypeStruct((_NP, 1), jnp.float32),
          jax.ShapeDtypeStruct((_NP, 1), jnp.float32),
          jax.ShapeDtypeStruct((_NP, 1), jnp.float32),
      ],
  )(acc, den, exs, h, bias, w, aa, elo)


def _tc_final_body(acc_ref, den_ref, exs_ref, h_ref, b_ref, out_ref):
  out_ref[...] = _combine_num_den(acc_ref, den_ref, exs_ref, h_ref, b_ref)


def _final(acc, den, exs, h, bias):
  g = 10
  b = _NP // g
  return pl.pallas_call(
      _tc_final_body,
      grid=(g,),
      in_specs=[
          pl.BlockSpec((2, b, _D), lambda i: (0, i, 0)),
          pl.BlockSpec((_NT, b, 1), lambda i: (0, i, 0)),
          pl.BlockSpec((b, 1), lambda i: (i, 0)),
          pl.BlockSpec((b, _D), lambda i: (i, 0)),
          pl.BlockSpec((8, _D), lambda i: (0, 0)),
      ],
      out_specs=pl.BlockSpec((b, _D), lambda i: (i, 0)),
      out_shape=jax.ShapeDtypeStruct((_NP, _D), jnp.float32),
  )(acc, den, exs, h, bias)


def _sc_edge_body(h_hbm, hs_hbm, hd_hbm, src_hbm, dst_hbm, eal_hbm,
                  acc_hbm, den_hbm,
                  hs_v, hd_v, den_v, src_v, dst_v, eal_v, ex_v, sdst_v,
                  rows_v, acc_s, sem_i0, sem_i1, sem_g, sem_s0, sem_s1):
  sem_i = (sem_i0, sem_i1)
  sem_s = (sem_s0, sem_s1)
  cid = lax.axis_index("c")
  sid = lax.axis_index("s")
  tid = cid * _NS + sid

  pltpu.sync_copy(hs_hbm, hs_v)
  pltpu.sync_copy(hd_hbm, hd_v)

  zero16 = jnp.zeros((16,), jnp.float32)

  def _zero_den(i, c):
    den_v[pl.ds(i * 16, 16)] = zero16
    return c
  lax.fori_loop(0, _NP // 16, _zero_den, 0)

  def _zero_rows(i, c):
    for q in range(_D // 16):
      rows_v[0, i, pl.ds(q * 16, 16)] = zero16
    return c
  lax.fori_loop(0, _CH, _zero_rows, 0)
  for k in range(_RPT // _CH):
    pltpu.sync_copy(rows_v.at[0],
                    acc_s.at[pl.ds(sid * _RPT + k * _CH, _CH)])
  plsc.subcore_barrier()

  def _idx_start(j, b):
    pltpu.async_copy(src_hbm.at[tid, j], src_v.at[b], sem_i[b])
    pltpu.async_copy(dst_hbm.at[tid, j], dst_v.at[b], sem_i[b])
    pltpu.async_copy(eal_hbm.at[tid, j], eal_v.at[b], sem_i[b])

  def _idx_wait(j, b):
    pltpu.make_async_copy(src_hbm.at[tid, j], src_v.at[b], sem_i[b]).wait()
    pltpu.make_async_copy(dst_hbm.at[tid, j], dst_v.at[b], sem_i[b]).wait()
    pltpu.make_async_copy(eal_hbm.at[tid, j], eal_v.at[b], sem_i[b]).wait()

  def _scatter_wait(b):
    pltpu.make_async_copy(rows_v.at[b], acc_s.at[sdst_v.at[b]],
                          sem_s[b]).wait()

  for b in range(2):
    _idx_start(b, b)

  def _pair(jj, c):
    for b in range(2):
      j = 2 * jj + b
      _idx_wait(j, b)

      @pl.when(jj > 0)
      def _():
        _scatter_wait(b)

      gat = pltpu.async_copy(h_hbm.at[src_v.at[b]], rows_v.at[b], sem_g)
      for q in range(_CH // 16):
        s16 = src_v[b, pl.ds(q * 16, 16)]
        d16 = dst_v[b, pl.ds(q * 16, 16)]
        sdst_v[b, pl.ds(q * 16, 16)] = d16
        al = (plsc.load_gather(hs_v, [s16]) + plsc.load_gather(hd_v, [d16])
              + eal_v[b, pl.ds(q * 16, 16)])
        al = jnp.where(al >= 0.0, al, 0.2 * al)
        e16 = jnp.exp(al)
        ex_v[b, pl.ds(q * 16, 16)] = e16
        plsc.addupdate_scatter(den_v, [d16], e16)
      gat.wait()

      @pl.when(jj < _NCHUNK // 2 - 1)
      def _():
        _idx_start(j + 2, b)

      def _scale(g, c2):
        e16 = ex_v[b, pl.ds(g * 16, 16)]
        for l in range(16):
          i = g * 16 + l
          e = e16[l]
          for q in range(_D // 16):
            rows_v[b, i, pl.ds(q * 16, 16)] = (
                rows_v[b, i, pl.ds(q * 16, 16)] * e)
        return c2
      lax.fori_loop(0, _CH // 16, _scale, 0)

      pltpu.async_copy(rows_v.at[b], acc_s.at[sdst_v.at[b]], sem_s[b],
                       add=True)
    return c

  lax.fori_loop(0, _NCHUNK // 2, _pair, 0)
  for b in range(2):
    _scatter_wait(b)

  pltpu.sync_copy(den_v, den_hbm.at[tid])
  plsc.subcore_barrier()
  for k in range(_RPT // _CH):
    r0 = sid * _RPT + k * _CH
    pltpu.sync_copy(acc_s.at[pl.ds(r0, _CH)], acc_hbm.at[cid, pl.ds(r0, _CH)])


_sc_edges = functools.partial(
    pl.kernel,
    out_type=[
        jax.ShapeDtypeStruct((_NC, _NP, _D), jnp.float32),
        jax.ShapeDtypeStruct((_NT, _NP), jnp.float32),
    ],
    mesh=plsc.VectorSubcoreMesh(core_axis_name="c", subcore_axis_name="s"),
    compiler_params=pltpu.CompilerParams(needs_layout_passes=False),
    scratch_types=[
        pltpu.VMEM((_NP,), jnp.float32),
        pltpu.VMEM((_NP,), jnp.float32),
        pltpu.VMEM((_NP,), jnp.float32),
        pltpu.VMEM((2, _CH), jnp.int32),
        pltpu.VMEM((2, _CH), jnp.int32),
        pltpu.VMEM((2, _CH), jnp.float32),
        pltpu.VMEM((2, _CH), jnp.float32),
        pltpu.VMEM((2, _CH), jnp.int32),
        pltpu.VMEM((2, _CH, _D), jnp.float32),
        pltpu.VMEM_SHARED((_NP, _D), jnp.float32),
        pltpu.SemaphoreType.DMA,
        pltpu.SemaphoreType.DMA,
        pltpu.SemaphoreType.DMA,
        pltpu.SemaphoreType.DMA,
        pltpu.SemaphoreType.DMA,
    ],
)(_sc_edge_body)


@jax.jit
def kernel(x, edge_index, edge_attr, W1, a_src1, a_dst1, We1, ae1, b1,
           W2, a_src2, a_dst2, We2, ae2, b2):
  f32 = jnp.float32

  xp = jnp.pad(x, ((0, _NP - _N), (0, 0)))
  src = jnp.pad(edge_index[0], (0, _EP - _E)).reshape(_NT, _NCHUNK, _CH)
  dst = jnp.pad(edge_index[1], (0, _EP - _E)).reshape(_NT, _NCHUNK, _CH)
  eap = edge_attr.reshape(_E // 8, _D)

  w12 = jnp.stack([We1 @ ae1, We2 @ ae2], axis=1)
  wbig = jnp.pad(jnp.kron(jnp.eye(8, dtype=f32), w12), ((0, 0), (0, _D - 16)))

  ealp, csum = _edge_alpha(eap, wbig)
  eal2 = ealp[:, :16].reshape(_E, 2)
  ea_mean = csum[0].reshape(8, _DE).sum(axis=0) / float(_E)
  elo = ea_mean @ w12

  def pack_eal(l):
    v = jnp.pad(eal2[:, l], (0, _EP - _E), constant_values=-1e30)
    return v.reshape(_NT, _NCHUNK, _CH)

  aa1 = jnp.pad(jnp.stack([a_src1, a_dst1]), ((0, 6), (0, 0)))
  aa2 = jnp.pad(jnp.stack([a_src2, a_dst2]), ((0, 6), (0, 0)))
  b1p = jnp.pad(b1.reshape(1, _D), ((0, 7), (0, 0)))
  b2p = jnp.pad(b2.reshape(1, _D), ((0, 7), (0, 0)))

  h1, hs1, hd1, exs1 = _dense1(xp, W1, aa1, elo[0:1])
  acc1, den1 = _sc_edges(h1, hs1.reshape(_NP), hd1.reshape(_NP),
                         src, dst, pack_eal(0))
  den1r = den1.reshape(_NT, _NP, 1)

  h2, hs2, hd2, exs2 = _mid(acc1, den1r, exs1, h1, b1p, W2, aa2, elo[1:2])
  acc2, den2 = _sc_edges(h2, hs2.reshape(_NP), hd2.reshape(_NP),
                         src, dst, pack_eal(1))
  den2r = den2.reshape(_NT, _NP, 1)

  out = _final(acc2, den2r, exs2, h2, b2p)
  return out[:_N]


# speedup vs baseline: 19.8849x; 1.5708x over previous
import functools

import jax
import jax.numpy as jnp
from jax import lax
from jax.experimental import pallas as pl
from jax.experimental.pallas import tpu as pltpu
from jax.experimental.pallas import tpu_sc as plsc

_N = 10000
_NP = 10240
_E = 320000
_D = 128
_DE = 16
_NC = 2
_NS = 16
_NT = _NC * _NS
_CH = 64
_NCHUNK = 160
_EP = _NT * _NCHUNK * _CH
_RPT = _NP // _NS


def _tc_edge_alpha_body(eap_ref, wbig_ref, eal_ref, csum_ref):
  i = pl.program_id(0)
  xb = eap_ref[...]
  eal_ref[...] = jnp.dot(xb, wbig_ref[...],
                         preferred_element_type=jnp.float32)
  part = jnp.broadcast_to(jnp.sum(xb, axis=0, keepdims=True), (8, _D))

  @pl.when(i == 0)
  def _():
    csum_ref[...] = part

  @pl.when(i > 0)
  def _():
    csum_ref[...] = csum_ref[...] + part


def _edge_alpha(eap, wbig):
  g = 10
  rows = eap.shape[0] // g
  return pl.pallas_call(
      _tc_edge_alpha_body,
      grid=(g,),
      in_specs=[
          pl.BlockSpec((rows, _D), lambda i: (i, 0)),
          pl.BlockSpec((_D, _D), lambda i: (0, 0)),
      ],
      out_specs=[
          pl.BlockSpec((rows, _D), lambda i: (i, 0)),
          pl.BlockSpec((8, _D), lambda i: (0, 0)),
      ],
      out_shape=[
          jax.ShapeDtypeStruct((eap.shape[0], _D), jnp.float32),
          jax.ShapeDtypeStruct((8, _D), jnp.float32),
      ],
  )(eap, wbig)


def _proj_tail(h, aa_ref, elo_ref, hs_ref, hd_ref, exs_ref):
  hs = jnp.sum(h * aa_ref[0:1, :], axis=1, keepdims=True)
  hd = jnp.sum(h * aa_ref[1:2, :], axis=1, keepdims=True)
  hs_ref[...] = hs
  hd_ref[...] = hd
  al = hs + hd + elo_ref[0]
  al = jnp.where(al >= 0.0, al, 0.2 * al)
  exs_ref[...] = jnp.exp(al)


def _tc_dense1_body(x_ref, w_ref, aa_ref, elo_ref, h_ref, hs_ref, hd_ref,
                    exs_ref):
  h = jnp.dot(x_ref[...], w_ref[...], preferred_element_type=jnp.float32)
  h_ref[...] = h
  _proj_tail(h, aa_ref, elo_ref, hs_ref, hd_ref, exs_ref)


def _dense1(x, w, aa, elo):
  g = 10
  b = _NP // g
  return pl.pallas_call(
      _tc_dense1_body,
      grid=(g,),
      in_specs=[
          pl.BlockSpec((b, _D), lambda i: (i, 0)),
          pl.BlockSpec((_D, _D), lambda i: (0, 0)),
          pl.BlockSpec((8, _D), lambda i: (0, 0)),
          pl.BlockSpec(memory_space=pltpu.SMEM),
      ],
      out_specs=[
          pl.BlockSpec((b, _D), lambda i: (i, 0)),
          pl.BlockSpec((b, 1), lambda i: (i, 0)),
          pl.BlockSpec((b, 1), lambda i: (i, 0)),
          pl.BlockSpec((b, 1), lambda i: (i, 0)),
      ],
      out_shape=[
          jax.ShapeDtypeStruct((_NP, _D), jnp.float32),
          jax.ShapeDtypeStruct((_NP, 1), jnp.float32),
          jax.ShapeDtypeStruct((_NP, 1), jnp.float32),
          jax.ShapeDtypeStruct((_NP, 1), jnp.float32),
      ],
  )(x, w, aa, elo)


def _combine_num_den(acc_ref, den_ref, exs_ref, h_ref, b_ref):
  exs = exs_ref[...]
  num = acc_ref[0] + acc_ref[1] + exs * h_ref[...]
  den = jnp.sum(den_ref[...], axis=0) + exs
  return num / jnp.maximum(den, 1e-16) + b_ref[0:1, :]


def _tc_mid_body(acc_ref, den_ref, exs_ref, h_ref, b_ref, w_ref, aa_ref,
                 elo_ref, h2_ref, hs_ref, hd_ref, exs_ref_o):
  x2 = jnp.maximum(_combine_num_den(acc_ref, den_ref, exs_ref, h_ref, b_ref),
                   0.0)
  h2 = jnp.dot(x2, w_ref[...], preferred_element_type=jnp.float32)
  h2_ref[...] = h2
  _proj_tail(h2, aa_ref, elo_ref, hs_ref, hd_ref, exs_ref_o)


def _mid(acc, den, exs, h, bias, w, aa, elo):
  g = 10
  b = _NP // g
  return pl.pallas_call(
      _tc_mid_body,
      grid=(g,),
      in_specs=[
          pl.BlockSpec((2, b, _D), lambda i: (0, i, 0)),
          pl.BlockSpec((_NT, b, 1), lambda i: (0, i, 0)),
          pl.BlockSpec((b, 1), lambda i: (i, 0)),
          pl.BlockSpec((b, _D), lambda i: (i, 0)),
          pl.BlockSpec((8, _D), lambda i: (0, 0)),
          pl.BlockSpec((_D, _D), lambda i: (0, 0)),
          pl.BlockSpec((8, _D), lambda i: (0, 0)),
          pl.BlockSpec(memory_space=pltpu.SMEM),
      ],
      out_specs=[
          pl.BlockSpec((b, _D), lambda i: (i, 0)),
          pl.BlockSpec((b, 1), lambda i: (i, 0)),
          pl.BlockSpec((b, 1), lambda i: (i, 0)),
          pl.BlockSpec((b, 1), lambda i: (i, 0)),
      ],
      out_shape=[
          jax.ShapeDtypeStruct((_NP, _D), jnp.float32),
          jax.ShapeDtypeStruct((_NP, 1), jnp.float32),
          jax.ShapeDtypeStruct((_NP, 1), jnp.float32),
          jax.ShapeDtypeStruct((_NP, 1), jnp.float32),
      ],
  )(acc, den, exs, h, bias, w, aa, elo)


def _tc_final_body(acc_ref, den_ref, exs_ref, h_ref, b_ref, out_ref):
  out_ref[...] = _combine_num_den(acc_ref, den_ref, exs_ref, h_ref, b_ref)


def _final(acc, den, exs, h, bias):
  g = 10
  b = _NP // g
  return pl.pallas_call(
      _tc_final_body,
      grid=(g,),
      in_specs=[
          pl.BlockSpec((2, b, _D), lambda i: (0, i, 0)),
          pl.BlockSpec((_NT, b, 1), lambda i: (0, i, 0)),
          pl.BlockSpec((b, 1), lambda i: (i, 0)),
          pl.BlockSpec((b, _D), lambda i: (i, 0)),
          pl.BlockSpec((8, _D), lambda i: (0, 0)),
      ],
      out_specs=pl.BlockSpec((b, _D), lambda i: (i, 0)),
      out_shape=jax.ShapeDtypeStruct((_NP, _D), jnp.float32),
  )(acc, den, exs, h, bias)


def _sc_edge_body(h_hbm, hs_hbm, hd_hbm, src_hbm, dst_hbm, eal_hbm,
                  acc_hbm, den_hbm,
                  hs_v, hd_v, den_v, src_v, dst_v, eal_v, ex_v, sdst_v,
                  rows_v, acc_s, sem_i0, sem_i1, sem_g, sem_s0, sem_s1):
  sem_i = (sem_i0, sem_i1)
  sem_s = (sem_s0, sem_s1)
  cid = lax.axis_index("c")
  sid = lax.axis_index("s")
  tid = cid * _NS + sid

  pltpu.sync_copy(hs_hbm, hs_v)
  pltpu.sync_copy(hd_hbm, hd_v)

  zero16 = jnp.zeros((16,), jnp.float32)

  def _zero_den(i, c):
    den_v[pl.ds(i * 16, 16)] = zero16
    return c
  lax.fori_loop(0, _NP // 16, _zero_den, 0)

  def _zero_rows(i, c):
    for q in range(_D // 16):
      rows_v[0, i, pl.ds(q * 16, 16)] = zero16
    return c
  lax.fori_loop(0, _CH, _zero_rows, 0)
  for k in range(_RPT // _CH):
    pltpu.sync_copy(rows_v.at[0],
                    acc_s.at[pl.ds(sid * _RPT + k * _CH, _CH)])
  plsc.subcore_barrier()

  def _idx_start(j, b):
    pltpu.async_copy(src_hbm.at[tid, j], src_v.at[b], sem_i[b])
    pltpu.async_copy(dst_hbm.at[tid, j], dst_v.at[b], sem_i[b])
    pltpu.async_copy(eal_hbm.at[tid, j], eal_v.at[b], sem_i[b])

  def _idx_wait(j, b):
    pltpu.make_async_copy(src_hbm.at[tid, j], src_v.at[b], sem_i[b]).wait()
    pltpu.make_async_copy(dst_hbm.at[tid, j], dst_v.at[b], sem_i[b]).wait()
    pltpu.make_async_copy(eal_hbm.at[tid, j], eal_v.at[b], sem_i[b]).wait()

  def _scatter_wait(b):
    pltpu.make_async_copy(rows_v.at[b], acc_s.at[sdst_v.at[b]],
                          sem_s[b]).wait()

  for b in range(2):
    _idx_start(b, b)

  def _pair(jj, c):
    for b in range(2):
      j = 2 * jj + b
      _idx_wait(j, b)

      @pl.when(jj > 0)
      def _():
        _scatter_wait(b)

      gat = pltpu.async_copy(h_hbm.at[src_v.at[b]], rows_v.at[b], sem_g)
      for q in range(_CH // 16):
        s16 = src_v[b, pl.ds(q * 16, 16)]
        d16 = dst_v[b, pl.ds(q * 16, 16)]
        sdst_v[b, pl.ds(q * 16, 16)] = d16
        al = (plsc.load_gather(hs_v, [s16]) + plsc.load_gather(hd_v, [d16])
              + eal_v[b, pl.ds(q * 16, 16)])
        al = jnp.where(al >= 0.0, al, 0.2 * al)
        e16 = jnp.exp(al)
        ex_v[b, pl.ds(q * 16, 16)] = e16
        plsc.addupdate_scatter(den_v, [d16], e16)
      gat.wait()

      @pl.when(jj < _NCHUNK // 2 - 1)
      def _():
        _idx_start(j + 2, b)

      def _scale(g, c2):
        e16 = ex_v[b, pl.ds(g * 16, 16)]
        for l in range(16):
          i = g * 16 + l
          e = e16[l]
          for q in range(_D // 16):
            rows_v[b, i, pl.ds(q * 16, 16)] = (
                rows_v[b, i, pl.ds(q * 16, 16)] * e)
        return c2
      lax.fori_loop(0, _CH // 16, _scale, 0)

      pltpu.async_copy(rows_v.at[b], acc_s.at[sdst_v.at[b]], sem_s[b],
                       add=True)
    return c

  lax.fori_loop(0, _NCHUNK // 2, _pair, 0)
  for b in range(2):
    _scatter_wait(b)

  pltpu.sync_copy(den_v, den_hbm.at[tid])
  plsc.subcore_barrier()
  for k in range(_RPT // _CH):
    r0 = sid * _RPT + k * _CH
    pltpu.sync_copy(acc_s.at[pl.ds(r0, _CH)], acc_hbm.at[cid, pl.ds(r0, _CH)])


_sc_edges = functools.partial(
    pl.kernel,
    out_type=[
        jax.ShapeDtypeStruct((_NC, _NP, _D), jnp.float32),
        jax.ShapeDtypeStruct((_NT, _NP), jnp.float32),
    ],
    mesh=plsc.VectorSubcoreMesh(core_axis_name="c", subcore_axis_name="s"),
    compiler_params=pltpu.CompilerParams(needs_layout_passes=False),
    scratch_types=[
        pltpu.VMEM((_NP,), jnp.float32),
        pltpu.VMEM((_NP,), jnp.float32),
        pltpu.VMEM((_NP,), jnp.float32),
        pltpu.VMEM((2, _CH), jnp.int32),
        pltpu.VMEM((2, _CH), jnp.int32),
        pltpu.VMEM((2, _CH), jnp.float32),
        pltpu.VMEM((2, _CH), jnp.float32),
        pltpu.VMEM((2, _CH), jnp.int32),
        pltpu.VMEM((2, _CH, _D), jnp.float32),
        pltpu.VMEM_SHARED((_NP, _D), jnp.float32),
        pltpu.SemaphoreType.DMA,
        pltpu.SemaphoreType.DMA,
        pltpu.SemaphoreType.DMA,
        pltpu.SemaphoreType.DMA,
        pltpu.SemaphoreType.DMA,
    ],
)(_sc_edge_body)


@jax.jit
def kernel(x, edge_index, edge_attr, W1, a_src1, a_dst1, We1, ae1, b1,
           W2, a_src2, a_dst2, We2, ae2, b2):
  f32 = jnp.float32

  xp = jnp.pad(x, ((0, _NP - _N), (0, 0)))
  pad_ids = _N + (jnp.arange(_EP - _E, dtype=jnp.int32) % (_NP - _N))
  src = jnp.concatenate([edge_index[0], pad_ids]).reshape(_NT, _NCHUNK, _CH)
  dst = jnp.concatenate([edge_index[1], pad_ids]).reshape(_NT, _NCHUNK, _CH)
  eap = edge_attr.reshape(_E // 8, _D)

  w12 = jnp.stack([We1 @ ae1, We2 @ ae2], axis=1)
  wbig = jnp.pad(jnp.kron(jnp.eye(8, dtype=f32), w12), ((0, 0), (0, _D - 16)))

  ealp, csum = _edge_alpha(eap, wbig)
  eal2 = ealp[:, :16].reshape(_E, 2)
  ea_mean = csum[0].reshape(8, _DE).sum(axis=0) / float(_E)
  elo = ea_mean @ w12

  def pack_eal(l):
    v = jnp.pad(eal2[:, l], (0, _EP - _E), constant_values=-1e30)
    return v.reshape(_NT, _NCHUNK, _CH)

  aa1 = jnp.pad(jnp.stack([a_src1, a_dst1]), ((0, 6), (0, 0)))
  aa2 = jnp.pad(jnp.stack([a_src2, a_dst2]), ((0, 6), (0, 0)))
  b1p = jnp.pad(b1.reshape(1, _D), ((0, 7), (0, 0)))
  b2p = jnp.pad(b2.reshape(1, _D), ((0, 7), (0, 0)))

  h1, hs1, hd1, exs1 = _dense1(xp, W1, aa1, elo[0:1])
  acc1, den1 = _sc_edges(h1, hs1.reshape(_NP), hd1.reshape(_NP),
                         src, dst, pack_eal(0))
  den1r = den1.reshape(_NT, _NP, 1)

  h2, hs2, hd2, exs2 = _mid(acc1, den1r, exs1, h1, b1p, W2, aa2, elo[1:2])
  acc2, den2 = _sc_edges(h2, hs2.reshape(_NP), hd2.reshape(_NP),
                         src, dst, pack_eal(1))
  den2r = den2.reshape(_NT, _NP, 1)

  out = _final(acc2, den2r, exs2, h2, b2p)
  return out[:_N]


# speedup vs baseline: 32.2472x; 1.6217x over previous
import functools

import jax
import jax.numpy as jnp
from jax import lax
from jax.experimental import pallas as pl
from jax.experimental.pallas import tpu as pltpu
from jax.experimental.pallas import tpu_sc as plsc

_N = 10000
_NP = 10240
_E = 320000
_D = 128
_DE = 16
_NC = 2
_NS = 16
_NT = _NC * _NS
_CH = 64
_EPT = 10240
_NCHUNK = _EPT // _CH
_RPT = _NP // _NS
_LAST = _E - (_NT - 1) * _EPT


def _pack_proj(h, aa_ref, hs_ref, hd_ref):
  b = h.shape[0]
  h3 = h.reshape(b // _D, _D, _D)
  hs_ref[...] = jnp.sum(h3 * aa_ref[0, :].reshape(1, 1, _D), axis=2)
  hd_ref[...] = jnp.sum(h3 * aa_ref[1, :].reshape(1, 1, _D), axis=2)


def _tc_dense1_body(x_ref, w_ref, aa_ref, h_ref, hs_ref, hd_ref):
  h = jnp.dot(x_ref[...], w_ref[...], preferred_element_type=jnp.float32)
  h_ref[...] = h
  _pack_proj(h, aa_ref, hs_ref, hd_ref)


def _dense1(x, w, aa):
  g = 10
  b = _NP // g
  return pl.pallas_call(
      _tc_dense1_body,
      grid=(g,),
      in_specs=[
          pl.BlockSpec((b, _D), lambda i: (i, 0)),
          pl.BlockSpec((_D, _D), lambda i: (0, 0)),
          pl.BlockSpec((8, _D), lambda i: (0, 0)),
      ],
      out_specs=[
          pl.BlockSpec((b, _D), lambda i: (i, 0)),
          pl.BlockSpec((b // _D, _D), lambda i: (i, 0)),
          pl.BlockSpec((b // _D, _D), lambda i: (i, 0)),
      ],
      out_shape=[
          jax.ShapeDtypeStruct((_NP, _D), jnp.float32),
          jax.ShapeDtypeStruct((_NP // _D, _D), jnp.float32),
          jax.ShapeDtypeStruct((_NP // _D, _D), jnp.float32),
      ],
  )(x, w, aa)


def _combine(acc_ref, den_ref, h_ref, aa_ref, b_ref, elo_ref):
  h = h_ref[...]
  bb = h.shape[0]
  als = jnp.sum(h * (aa_ref[0:1, :] + aa_ref[1:2, :]), axis=1,
                keepdims=True) + elo_ref[0]
  als = jnp.where(als >= 0.0, als, 0.2 * als)
  exs = jnp.exp(als)
  dsum = jnp.sum(den_ref[...], axis=0)
  den = lax.broadcast_in_dim(dsum, (bb // _D, _D, _D),
                             (0, 1)).reshape(bb, _D)
  den = jnp.maximum(den + exs, 1e-16)
  num = acc_ref[0] + acc_ref[1] + exs * h
  return num / den + b_ref[0:1, :]


def _tc_mid_body(acc_ref, den_ref, h_ref, aa1_ref, b_ref, w_ref, aa2_ref,
                 elo_ref, h2_ref, hs_ref, hd_ref):
  x2 = jnp.maximum(_combine(acc_ref, den_ref, h_ref, aa1_ref, b_ref, elo_ref),
                   0.0)
  h2 = jnp.dot(x2, w_ref[...], preferred_element_type=jnp.float32)
  h2_ref[...] = h2
  _pack_proj(h2, aa2_ref, hs_ref, hd_ref)


def _mid(acc, den, h, aa1, bias, w, aa2, elo):
  g = 10
  b = _NP // g
  return pl.pallas_call(
      _tc_mid_body,
      grid=(g,),
      in_specs=[
          pl.BlockSpec((2, b, _D), lambda i: (0, i, 0)),
          pl.BlockSpec((_NC, b // _D, _D), lambda i: (0, i, 0)),
          pl.BlockSpec((b, _D), lambda i: (i, 0)),
          pl.BlockSpec((8, _D), lambda i: (0, 0)),
          pl.BlockSpec((8, _D), lambda i: (0, 0)),
          pl.BlockSpec((_D, _D), lambda i: (0, 0)),
          pl.BlockSpec((8, _D), lambda i: (0, 0)),
          pl.BlockSpec(memory_space=pltpu.SMEM),
      ],
      out_specs=[
          pl.BlockSpec((b, _D), lambda i: (i, 0)),
          pl.BlockSpec((b // _D, _D), lambda i: (i, 0)),
          pl.BlockSpec((b // _D, _D), lambda i: (i, 0)),
      ],
      out_shape=[
          jax.ShapeDtypeStruct((_NP, _D), jnp.float32),
          jax.ShapeDtypeStruct((_NP // _D, _D), jnp.float32),
          jax.ShapeDtypeStruct((_NP // _D, _D), jnp.float32),
      ],
  )(acc, den, h, aa1, bias, w, aa2, elo)


def _tc_final_body(acc_ref, den_ref, h_ref, aa_ref, b_ref, elo_ref, out_ref):
  out_ref[...] = _combine(acc_ref, den_ref, h_ref, aa_ref, b_ref, elo_ref)


def _final(acc, den, h, aa, bias, elo):
  g = 10
  b = _NP // g
  return pl.pallas_call(
      _tc_final_body,
      grid=(g,),
      in_specs=[
          pl.BlockSpec((2, b, _D), lambda i: (0, i, 0)),
          pl.BlockSpec((_NC, b // _D, _D), lambda i: (0, i, 0)),
          pl.BlockSpec((b, _D), lambda i: (i, 0)),
          pl.BlockSpec((8, _D), lambda i: (0, 0)),
          pl.BlockSpec((8, _D), lambda i: (0, 0)),
          pl.BlockSpec(memory_space=pltpu.SMEM),
      ],
      out_specs=pl.BlockSpec((b, _D), lambda i: (i, 0)),
      out_shape=jax.ShapeDtypeStruct((_NP, _D), jnp.float32),
  )(acc, den, h, aa, bias, elo)


def _sc_edge_body(h_hbm, hs_hbm, hd_hbm, ei_hbm, ea_hbm, w_hbm,
                  acc_hbm, den_hbm, easum_hbm,
                  hs_v, hd_v, zer_v, src_v, dst_v, ex_v, sdst_v, ea_v, wsp_v,
                  rows_v, acc_s, den_s, sem_i0, sem_i1, sem_g, sem_s0, sem_s1, sem_e):
  sem_i = (sem_i0, sem_i1)
  sem_s = (sem_s0, sem_s1)
  cid = lax.axis_index("c")
  sid = lax.axis_index("s")
  tid = cid * _NS + sid
  base = tid * _EPT
  nch = jnp.where(tid == _NT - 1, _LAST // _CH, _NCHUNK)

  pltpu.sync_copy(hs_hbm, hs_v)
  pltpu.sync_copy(hd_hbm, hd_v)
  pltpu.sync_copy(w_hbm, wsp_v)

  zero16 = jnp.zeros((16,), jnp.float32)

  def _zero_den(i, c):
    zer_v[pl.ds(i * 16, 16)] = zero16
    return c
  lax.fori_loop(0, _RPT // 16, _zero_den, 0)
  pltpu.sync_copy(zer_v, den_s.at[pl.ds(sid * _RPT, _RPT)])

  def _zero_rows(i, c):
    for q in range(_D // 16):
      rows_v[0, i, pl.ds(q * 16, 16)] = zero16
    return c
  lax.fori_loop(0, _CH, _zero_rows, 0)
  for k in range(_RPT // _CH):
    pltpu.sync_copy(rows_v.at[0],
                    acc_s.at[pl.ds(sid * _RPT + k * _CH, _CH)])
  plsc.subcore_barrier()

  def _idx_start(j, b):
    off = base + j * _CH
    pltpu.async_copy(ei_hbm.at[0, pl.ds(off, _CH)],
                     src_v.at[pl.ds(b * _CH, _CH)], sem_i[b])
    pltpu.async_copy(ei_hbm.at[1, pl.ds(off, _CH)],
                     dst_v.at[pl.ds(b * _CH, _CH)], sem_i[b])

  def _idx_wait(j, b):
    off = base + j * _CH
    pltpu.make_async_copy(ei_hbm.at[0, pl.ds(off, _CH)],
                          src_v.at[pl.ds(b * _CH, _CH)], sem_i[b]).wait()
    pltpu.make_async_copy(ei_hbm.at[1, pl.ds(off, _CH)],
                          dst_v.at[pl.ds(b * _CH, _CH)], sem_i[b]).wait()

  def _ea_start(j):
    off8 = pl.multiple_of((base + j * _CH) // 8, 8)
    pltpu.async_copy(ea_hbm.at[pl.ds(off8, 8)], ea_v, sem_e)

  def _ea_wait(j):
    off8 = pl.multiple_of((base + j * _CH) // 8, 8)
    pltpu.make_async_copy(ea_hbm.at[pl.ds(off8, 8)], ea_v, sem_e).wait()

  def _scatter_wait(b):
    pltpu.make_async_copy(rows_v.at[b], acc_s.at[sdst_v.at[b]],
                          sem_s[b]).wait()
    pltpu.make_async_copy(ex_v.at[pl.ds(b * _CH, _CH)],
                          den_s.at[sdst_v.at[b]], sem_s[b]).wait()

  _ea_start(0)
  for b in range(2):
    _idx_start(b, b)

  iota16 = lax.iota(jnp.int32, 16)

  def _pair(jj, es):
    for b in range(2):
      j = 2 * jj + b
      _idx_wait(j, b)
      _ea_wait(j)

      @pl.when(jj > 0)
      def _():
        _scatter_wait(b)

      gat = pltpu.async_copy(h_hbm.at[src_v.at[pl.ds(b * _CH, _CH)]],
                             rows_v.at[b], sem_g)

      def _esum(rr, a):
        for jj8 in range(8):
          a = a + ea_v[rr, pl.ds(jj8 * _DE, _DE)]
        return a
      es = lax.fori_loop(0, 8, _esum, es)

      for q in range(_CH // 16):
        s16 = src_v[pl.ds(b * _CH + q * 16, 16)]
        d16 = dst_v[pl.ds(b * _CH + q * 16, 16)]
        sdst_v[b, pl.ds(q * 16, 16)] = d16
        al = plsc.load_gather(hs_v, [s16]) + plsc.load_gather(hd_v, [d16])
        row16 = (iota16 >> 3) + (2 * q)
        colb16 = (iota16 & 7) << 4

        def _kstep(k, a):
          wk = wsp_v[pl.ds(k * 16, 16)]
          return a + plsc.load_gather(ea_v, [row16, colb16 + k]) * wk
        al = lax.fori_loop(0, _DE, _kstep, al)
        al = jnp.where(al >= 0.0, al, 0.2 * al)
        e16 = jnp.exp(al)
        ex_v[pl.ds(b * _CH + q * 16, 16)] = e16
      pltpu.async_copy(ex_v.at[pl.ds(b * _CH, _CH)], den_s.at[sdst_v.at[b]],
                       sem_s[b], add=True)
      gat.wait()

      @pl.when(j + 1 < nch)
      def _():
        _ea_start(j + 1)

      @pl.when(j + 2 < nch)
      def _():
        _idx_start(j + 2, b)

      def _scale(g, c2):
        e16 = ex_v[pl.ds(b * _CH + g * 16, 16)]
        for l in range(16):
          i = g * 16 + l
          e = e16[l]
          for q in range(_D // 16):
            rows_v[b, i, pl.ds(q * 16, 16)] = (
                rows_v[b, i, pl.ds(q * 16, 16)] * e)
        return c2
      lax.fori_loop(0, _CH // 16, _scale, 0)

      pltpu.async_copy(rows_v.at[b], acc_s.at[sdst_v.at[b]], sem_s[b],
                       add=True)
    return es

  es = lax.fori_loop(0, nch // 2, _pair, jnp.zeros((_DE,), jnp.float32))
  for b in range(2):
    _scatter_wait(b)

  ex_v[pl.ds(0, _DE)] = es
  for z in range(1, _D // _DE):
    ex_v[pl.ds(z * _DE, _DE)] = jnp.zeros((_DE,), jnp.float32)
  pltpu.sync_copy(ex_v, easum_hbm.at[tid])

  plsc.subcore_barrier()
  pltpu.sync_copy(den_s.at[pl.ds(sid * _RPT, _RPT)],
                  den_hbm.at[cid, pl.ds(sid * _RPT, _RPT)])
  for k in range(_RPT // _CH):
    r0 = sid * _RPT + k * _CH
    pltpu.sync_copy(acc_s.at[pl.ds(r0, _CH)], acc_hbm.at[cid, pl.ds(r0, _CH)])


_sc_edges = functools.partial(
    pl.kernel,
    out_type=[
        jax.ShapeDtypeStruct((_NC, _NP, _D), jnp.float32),
        jax.ShapeDtypeStruct((_NC, _NP), jnp.float32),
        jax.ShapeDtypeStruct((_NT, _D), jnp.float32),
    ],
    mesh=plsc.VectorSubcoreMesh(core_axis_name="c", subcore_axis_name="s"),
    compiler_params=pltpu.CompilerParams(needs_layout_passes=False),
    scratch_types=[
        pltpu.VMEM((_NP,), jnp.float32),
        pltpu.VMEM((_NP,), jnp.float32),
        pltpu.VMEM((_RPT,), jnp.float32),
        pltpu.VMEM((2 * _CH,), jnp.int32),
        pltpu.VMEM((2 * _CH,), jnp.int32),
        pltpu.VMEM((2 * _CH,), jnp.float32),
        pltpu.VMEM((2, _CH), jnp.int32),
        pltpu.VMEM((8, _D), jnp.float32),
        pltpu.VMEM((_DE * 16,), jnp.float32),
        pltpu.VMEM((2, _CH, _D), jnp.float32),
        pltpu.VMEM_SHARED((_NP, _D), jnp.float32),
        pltpu.VMEM_SHARED((_NP,), jnp.float32),
        pltpu.SemaphoreType.DMA,
        pltpu.SemaphoreType.DMA,
        pltpu.SemaphoreType.DMA,
        pltpu.SemaphoreType.DMA,
        pltpu.SemaphoreType.DMA,
        pltpu.SemaphoreType.DMA,
    ],
)(_sc_edge_body)


@jax.jit
def kernel(x, edge_index, edge_attr, W1, a_src1, a_dst1, We1, ae1, b1,
           W2, a_src2, a_dst2, We2, ae2, b2):
  f32 = jnp.float32

  xp = jnp.pad(x, ((0, _NP - _N), (0, 0)))
  w12 = jnp.stack([We1 @ ae1, We2 @ ae2], axis=1)

  aa1 = jnp.pad(jnp.stack([a_src1, a_dst1]), ((0, 6), (0, 0)))
  aa2 = jnp.pad(jnp.stack([a_src2, a_dst2]), ((0, 6), (0, 0)))
  b1p = jnp.pad(b1.reshape(1, _D), ((0, 7), (0, 0)))
  b2p = jnp.pad(b2.reshape(1, _D), ((0, 7), (0, 0)))

  h1, hs1, hd1 = _dense1(xp, W1, aa1)
  eaw = edge_attr.reshape(_E // 8, _D)
  wsp = jnp.broadcast_to(w12.T.reshape(2, _DE, 1), (2, _DE, 16)).reshape(2, -1)
  acc1, den1, easum = _sc_edges(h1, hs1.reshape(_NP), hd1.reshape(_NP),
                                edge_index, eaw, wsp[0])
  elo = (easum[:, :_DE].sum(axis=0) / float(_E)) @ w12

  den1r = den1.reshape(_NC, _NP // _D, _D)
  h2, hs2, hd2 = _mid(acc1, den1r, h1, aa1, b1p, W2, aa2, elo[0:1])
  acc2, den2, _ = _sc_edges(h2, hs2.reshape(_NP), hd2.reshape(_NP),
                            edge_index, eaw, wsp[1])

  out = _final(acc2, den2.reshape(_NC, _NP // _D, _D), h2, aa2, b2p,
               elo[1:2])
  return out[:_N]
